# baseline (speedup 1.0000x reference)
import functools

import jax
import jax.numpy as jnp
from jax import lax
from jax.experimental import pallas as pl
from jax.experimental.pallas import tpu as pltpu
from jax.experimental.pallas import tpu_sc as plsc

_B, _N = 4096, 50
_MAX_OUT = 4
_D = 64
_ROW = _MAX_OUT * _D
_NUM_EMB = 8

_NW = 32
_RW = (_B * _N) // _NW
_CHUNK = 128
_NCH = _RW // _CHUNK


def _sc_body(subs_hbm, mask_hbm, tab_hbm, out_hbm, cnt_hbm,
             sub_v, msk_v, idx2, cnt_v, rows_a, rows_b, sem_a, sem_b):
    wid = lax.axis_index("s") * 2 + lax.axis_index("c")
    base = wid * _RW

    pltpu.sync_copy(subs_hbm.at[pl.ds(base, _RW)], sub_v)
    pltpu.sync_copy(mask_hbm.at[pl.ds(base, _RW)], msk_v)

    def compute_grp(i, _):
        s = sub_v[pl.ds(i * 16, 16)]
        m = msk_v[pl.ds(i * 16, 16)]
        ids = jnp.minimum(jnp.maximum(s + 1, 0), _NUM_EMB - 1)
        eff = jnp.where(m > 0, ids, _NUM_EMB)
        c = i // 8
        o = (i % 8) * 16
        idx2[c, pl.ds(o, 16)] = eff
        cnt_v[pl.ds(i * 16, 16)] = m.astype(jnp.float32)
        return 0

    lax.fori_loop(0, _RW // 16, compute_grp, 0)
    pltpu.sync_copy(cnt_v, cnt_hbm.at[pl.ds(base, _RW)])

    def do_gather(c, buf, sem):
        return pltpu.async_copy(tab_hbm.at[idx2.at[c]], buf, sem)

    g0 = do_gather(0, rows_a, sem_a)

    def chunk(c, _):
        even = c % 2 == 0

        @pl.when(even)
        def _():
            pltpu.async_copy(tab_hbm.at[idx2.at[c + 1]], rows_b, sem_b)
            pltpu.make_async_copy(tab_hbm.at[idx2.at[c]], rows_a, sem_a).wait()
            pltpu.sync_copy(rows_a, out_hbm.at[pl.ds(base + c * _CHUNK, _CHUNK)])

        @pl.when(jnp.logical_not(even))
        def _():
            pltpu.async_copy(tab_hbm.at[idx2.at[c + 1]], rows_a, sem_a)
            pltpu.make_async_copy(tab_hbm.at[idx2.at[c]], rows_b, sem_b).wait()
            pltpu.sync_copy(rows_b, out_hbm.at[pl.ds(base + c * _CHUNK, _CHUNK)])

        return 0

    lax.fori_loop(0, _NCH - 1, chunk, 0)
    c_last = _NCH - 1
    pltpu.make_async_copy(tab_hbm.at[idx2.at[c_last]], rows_b, sem_b).wait()
    pltpu.sync_copy(rows_b, out_hbm.at[pl.ds(base + c_last * _CHUNK, _CHUNK)])


def kernel(subs, mask, embed_table):
    subs_flat = subs.reshape(_B * _N)
    mask_flat = mask.astype(jnp.int32).reshape(_B * _N)
    tab_aug = jnp.zeros((16, _ROW), jnp.float32)
    tab_aug = tab_aug.at[:_NUM_EMB, :_D].set(embed_table)

    mesh = plsc.VectorSubcoreMesh(core_axis_name="c", subcore_axis_name="s")
    run = functools.partial(
        pl.kernel, mesh=mesh,
        out_type=[
            jax.ShapeDtypeStruct((_B * _N, _ROW), jnp.float32),
            jax.ShapeDtypeStruct((_B * _N,), jnp.float32),
        ],
        scratch_types=[
            pltpu.VMEM((_RW,), jnp.int32),
            pltpu.VMEM((_RW,), jnp.int32),
            pltpu.VMEM((_NCH, _CHUNK), jnp.int32),
            pltpu.VMEM((_RW,), jnp.float32),
            pltpu.VMEM((_CHUNK, _ROW), jnp.float32),
            pltpu.VMEM((_CHUNK, _ROW), jnp.float32),
            pltpu.SemaphoreType.DMA,
            pltpu.SemaphoreType.DMA,
        ],
    )(_sc_body)
    buf, cnt = run(subs_flat, mask_flat, tab_aug)
    return buf.reshape(_B, _N, _MAX_OUT, _D), cnt.reshape(_B, _N)


# speedup vs baseline: 14.5958x; 14.5958x over previous
import jax
import jax.numpy as jnp
from jax.experimental import pallas as pl
from jax.experimental.pallas import tpu as pltpu

_B, _N = 4096, 50
_MAX_OUT = 4
_D = 64
_NUM_EMB = 8
_BB = 64
_ROWS = _BB * _N


def _emb_kernel(subs_ref, mask_ref, tab_ref, buf_ref, cnt_ref):
    subs = subs_ref[...]
    mf = mask_ref[...].astype(jnp.float32)
    ids = jnp.clip(subs + 1, 0, _NUM_EMB - 1)
    k_iota = jax.lax.broadcasted_iota(jnp.int32, (1, 1, _NUM_EMB), 2)
    oh = (ids[..., None] == k_iota).astype(jnp.float32) * mf[..., None]
    prim = jax.lax.dot_general(
        oh.reshape(_ROWS, _NUM_EMB), tab_ref[...],
        (((1,), (0,)), ((), ())), preferred_element_type=jnp.float32)
    buf_ref[:, :, 0:_D] = prim.reshape(_BB, _N, _D)
    buf_ref[:, :, _D:] = jnp.zeros((_BB, _N, (_MAX_OUT - 1) * _D), jnp.float32)
    cnt_ref[...] = mf


def kernel(subs, mask, embed_table):
    mask_i = mask.astype(jnp.int32)
    grid = (_B // _BB,)
    buf, cnt = pl.pallas_call(
        _emb_kernel,
        grid=grid,
        in_specs=[
            pl.BlockSpec((_BB, _N), lambda i: (i, 0)),
            pl.BlockSpec((_BB, _N), lambda i: (i, 0)),
            pl.BlockSpec((_NUM_EMB, _D), lambda i: (0, 0)),
        ],
        out_specs=[
            pl.BlockSpec((_BB, _N, _MAX_OUT * _D), lambda i: (i, 0, 0)),
            pl.BlockSpec((_BB, _N), lambda i: (i, 0)),
        ],
        out_shape=[
            jax.ShapeDtypeStruct((_B, _N, _MAX_OUT * _D), jnp.float32),
            jax.ShapeDtypeStruct((_B, _N), jnp.float32),
        ],
        compiler_params=pltpu.CompilerParams(
            dimension_semantics=("parallel",)),
    )(subs, mask_i, embed_table)
    return buf.reshape(_B, _N, _MAX_OUT, _D), cnt
